# baseline (speedup 1.0000x reference)
import functools

import jax
import jax.numpy as jnp
from jax import lax
from jax.experimental import pallas as pl
from jax.experimental.pallas import tpu as pltpu
from jax.experimental.pallas import tpu_sc as plsc

_NC = 2
_NS = 16
_NW = _NC * _NS
_LANES = 16


def _make_hist(n_pad: int, chunk: int, hw: int, w: int):
    per_tile = hw // _NW
    assert hw % _NW == 0 and chunk % _LANES == 0 and n_pad % chunk == 0

    @functools.partial(
        pl.kernel,
        mesh=plsc.VectorSubcoreMesh(core_axis_name="c", subcore_axis_name="s"),
        compiler_params=pltpu.CompilerParams(needs_layout_passes=False),
        out_type=jax.ShapeDtypeStruct((hw,), jnp.float32),
        scratch_types=[
            pltpu.VMEM((chunk * 2,), jnp.int32),
            pltpu.VMEM((per_tile,), jnp.float32),
        ],
    )
    def hist(dots_hbm, out_hbm, dots_v, hist_v):
        cid = lax.axis_index("c")
        sid = lax.axis_index("s")
        wid = sid * _NC + cid
        base = wid * per_tile

        zeros16 = jnp.zeros((_LANES,), jnp.float32)

        def zbody(k, carry):
            hist_v[pl.ds(k * _LANES, _LANES)] = zeros16
            return carry

        lax.fori_loop(0, per_tile // _LANES, zbody, 0)

        iota2 = lax.iota(jnp.int32, _LANES) * 2
        ones16 = jnp.ones((_LANES,), jnp.float32)

        for c in range(n_pad // chunk):
            pltpu.sync_copy(dots_hbm.at[pl.ds(c * chunk * 2, chunk * 2)], dots_v)

            def body(i, carry):
                off = i * (2 * _LANES) + iota2
                xs = plsc.load_gather(dots_v, [off])
                ys = plsc.load_gather(dots_v, [off + 1])
                local = ys * w + xs - base
                mask = (local >= 0) & (local < per_tile)
                safe = jnp.minimum(jnp.maximum(local, 0), per_tile - 1)
                plsc.addupdate_scatter(hist_v, [safe], ones16, mask=mask)
                return carry

            lax.fori_loop(0, chunk // _LANES, body, 0)

        pltpu.sync_copy(hist_v, out_hbm.at[pl.ds(base, per_tile)])

    return hist


def _sqdiff_body(img_ref, rew_ref, s_ref):
    d = img_ref[0] - rew_ref[0]

    @pl.when(pl.program_id(0) == 0)
    def _():
        s_ref[...] = jnp.zeros_like(s_ref)

    s_ref[...] += d * d


def _wsum_body(s_ref, cnt_ref, tot_ref):
    tot_ref[0, 0] = jnp.sum(s_ref[...] * cnt_ref[...])


def kernel(image, image_rewrite, dot_list_format):
    c, h, w = image.shape
    n = dot_list_format.shape[0]
    hw = h * w

    chunk = 10000
    if chunk % _LANES:
        chunk = ((chunk // _LANES) + 1) * _LANES
    n_pad = ((n + chunk - 1) // chunk) * chunk
    dots = dot_list_format
    if n_pad != n:
        fill = jnp.concatenate(
            [
                jnp.zeros((n_pad - n, 1), jnp.int32),
                jnp.full((n_pad - n, 1), h, jnp.int32),
            ],
            axis=1,
        )
        dots = jnp.concatenate([dots, fill], axis=0)
    dots_flat = dots.reshape(n_pad * 2)

    counts = _make_hist(n_pad, chunk, hw, w)(dots_flat)
    counts2d = counts.reshape(h, w)

    s = pl.pallas_call(
        _sqdiff_body,
        grid=(c,),
        in_specs=[
            pl.BlockSpec((1, h, w), lambda i: (i, 0, 0)),
            pl.BlockSpec((1, h, w), lambda i: (i, 0, 0)),
        ],
        out_specs=pl.BlockSpec((h, w), lambda i: (0, 0)),
        out_shape=jax.ShapeDtypeStruct((h, w), jnp.float32),
    )(image, image_rewrite)

    tot = pl.pallas_call(
        _wsum_body,
        in_specs=[
            pl.BlockSpec((h, w), lambda: (0, 0)),
            pl.BlockSpec((h, w), lambda: (0, 0)),
        ],
        out_specs=pl.BlockSpec(memory_space=pltpu.SMEM),
        out_shape=jax.ShapeDtypeStruct((1, 1), jnp.float32),
    )(s, counts2d)

    return tot[0, 0] / jnp.float32(n)


# speedup vs baseline: 1.0599x; 1.0599x over previous
import functools

import jax
import jax.numpy as jnp
from jax import lax
from jax.experimental import pallas as pl
from jax.experimental.pallas import tpu as pltpu
from jax.experimental.pallas import tpu_sc as plsc

_NC = 2
_NS = 16
_NW = _NC * _NS
_LANES = 16


def _make_hist(n_pad: int, chunk: int, hw: int, w: int):
    per_tile = hw // _NW
    assert hw % _NW == 0 and chunk % _LANES == 0 and n_pad % chunk == 0

    @functools.partial(
        pl.kernel,
        mesh=plsc.VectorSubcoreMesh(core_axis_name="c", subcore_axis_name="s"),
        compiler_params=pltpu.CompilerParams(needs_layout_passes=False),
        out_type=jax.ShapeDtypeStruct((hw,), jnp.float32),
        scratch_types=[
            pltpu.VMEM((chunk * 2,), jnp.int32),
            pltpu.VMEM((per_tile,), jnp.float32),
        ],
    )
    def hist(dots_hbm, out_hbm, dots_v, hist_v):
        cid = lax.axis_index("c")
        sid = lax.axis_index("s")
        wid = sid * _NC + cid
        base = wid * per_tile

        zeros16 = jnp.zeros((_LANES,), jnp.float32)

        def zbody(k, carry):
            hist_v[pl.ds(k * _LANES, _LANES)] = zeros16
            return carry

        lax.fori_loop(0, per_tile // _LANES, zbody, 0)

        iota2 = lax.iota(jnp.int32, _LANES) * 2
        ones16 = jnp.ones((_LANES,), jnp.float32)

        for c in range(n_pad // chunk):
            pltpu.sync_copy(dots_hbm.at[pl.ds(c * chunk * 2, chunk * 2)], dots_v)

            def body(i, carry):
                off = i * (2 * _LANES) + iota2
                xs = plsc.load_gather(dots_v, [off])
                ys = plsc.load_gather(dots_v, [off + 1])
                local = ys * w + xs - base
                mask = (local >= 0) & (local < per_tile)
                safe = jnp.minimum(jnp.maximum(local, 0), per_tile - 1)
                plsc.addupdate_scatter(hist_v, [safe], ones16, mask=mask)
                return carry

            lax.fori_loop(0, chunk // _LANES, body, 0)

        pltpu.sync_copy(hist_v, out_hbm.at[pl.ds(base, per_tile)])

    return hist


def _sqdiff_body(img_ref, rew_ref, s_ref):
    d = img_ref[...] - rew_ref[...]
    s_ref[...] = jnp.sum(d * d, axis=0)


def _wsum_body(s_ref, cnt_ref, tot_ref):
    tot_ref[0, 0] = jnp.sum(s_ref[...] * cnt_ref[...])


def kernel(image, image_rewrite, dot_list_format):
    c, h, w = image.shape
    n = dot_list_format.shape[0]
    hw = h * w

    chunk = 10000
    if chunk % _LANES:
        chunk = ((chunk // _LANES) + 1) * _LANES
    n_pad = ((n + chunk - 1) // chunk) * chunk
    dots = dot_list_format
    if n_pad != n:
        fill = jnp.concatenate(
            [
                jnp.zeros((n_pad - n, 1), jnp.int32),
                jnp.full((n_pad - n, 1), h, jnp.int32),
            ],
            axis=1,
        )
        dots = jnp.concatenate([dots, fill], axis=0)
    dots_flat = dots.reshape(n_pad * 2)

    counts = _make_hist(n_pad, chunk, hw, w)(dots_flat)
    counts2d = counts.reshape(h, w)

    br = 64
    s = pl.pallas_call(
        _sqdiff_body,
        grid=(h // br,),
        in_specs=[
            pl.BlockSpec((c, br, w), lambda i: (0, i, 0)),
            pl.BlockSpec((c, br, w), lambda i: (0, i, 0)),
        ],
        out_specs=pl.BlockSpec((br, w), lambda i: (i, 0)),
        out_shape=jax.ShapeDtypeStruct((h, w), jnp.float32),
    )(image, image_rewrite)

    tot = pl.pallas_call(
        _wsum_body,
        in_specs=[
            pl.BlockSpec((h, w), lambda: (0, 0)),
            pl.BlockSpec((h, w), lambda: (0, 0)),
        ],
        out_specs=pl.BlockSpec(memory_space=pltpu.SMEM),
        out_shape=jax.ShapeDtypeStruct((1, 1), jnp.float32),
    )(s, counts2d)

    return tot[0, 0] / jnp.float32(n)
